# baseline (speedup 1.0000x reference)
import functools

import jax
import jax.numpy as jnp
from jax import lax
from jax.experimental import pallas as pl
from jax.experimental.pallas import tpu as pltpu
from jax.experimental.pallas import tpu_sc as plsc

_L = 16
_NC = 2
_NS = 16
_IDX_CHUNK = 128
_BL = 32768
_SPLIT = 196608
_CL = 512


@functools.lru_cache(maxsize=None)
def _build_tc_scores(V, D, bl, split):
    n_skip = split // bl
    nb = (V - split + bl - 1) // bl

    def body(tu_ref, ti_ref, w_ref, ou_ref, oi_ref):
        w = w_ref[...]
        pu = jax.lax.dot_general(
            w[:D], tu_ref[...], (((0,), (0,)), ((), ())),
            preferred_element_type=jnp.float32)
        ou_ref[...] = pu.reshape(ou_ref.shape)
        pi = jax.lax.dot_general(
            w[D:], ti_ref[...], (((0,), (0,)), ((), ())),
            preferred_element_type=jnp.float32)
        oi_ref[...] = pi.reshape(oi_ref.shape)

    return pl.pallas_call(
        body,
        grid=(nb,),
        in_specs=[
            pl.BlockSpec((D, bl), lambda i: (0, i + n_skip)),
            pl.BlockSpec((D, bl), lambda i: (0, i + n_skip)),
            pl.BlockSpec((2 * D, 1), lambda i: (0, 0)),
        ],
        out_specs=[
            pl.BlockSpec((bl,), lambda i: (i,)),
            pl.BlockSpec((bl,), lambda i: (i,)),
        ],
        out_shape=[
            jax.ShapeDtypeStruct((V - split,), jnp.float32),
            jax.ShapeDtypeStruct((V - split,), jnp.float32),
        ],
        compiler_params=pltpu.CompilerParams(
            dimension_semantics=("parallel",)),
    )


@functools.lru_cache(maxsize=None)
def _build_sc_scores(V, D, split):
    nw = _NC * _NS
    vpw = split // nw
    nch = vpw // _CL

    mesh = plsc.VectorSubcoreMesh(core_axis_name="c", subcore_axis_name="s")

    @functools.partial(
        pl.kernel,
        mesh=mesh,
        compiler_params=pltpu.CompilerParams(needs_layout_passes=False),
        out_type=[
            jax.ShapeDtypeStruct((split,), jnp.float32),
            jax.ShapeDtypeStruct((split,), jnp.float32),
        ],
        scratch_types=[
            pltpu.VMEM((2, D, _CL), jnp.float32),
            pltpu.VMEM((2, D, _CL), jnp.float32),
            pltpu.VMEM((2 * D,), jnp.float32),
            pltpu.VMEM((vpw,), jnp.float32),
            pltpu.VMEM((vpw,), jnp.float32),
            pltpu.SemaphoreType.DMA,
            pltpu.SemaphoreType.DMA,
        ],
    )
    def sc_scores(tu_hbm, ti_hbm, w_hbm, su_hbm, si_hbm,
                  tu_b, ti_b, w_v, su_v, si_v, sem_u, sem_i):
        wid = lax.axis_index("s") * _NC + lax.axis_index("c")
        base = wid * vpw

        pltpu.sync_copy(w_hbm, w_v)
        wh = [w_v[pl.ds(h * _L, _L)] for h in range(2 * D // _L)]

        def start(ch, slot):
            src = pl.ds(base + ch * _CL, _CL)
            return (pltpu.async_copy(tu_hbm.at[:, src], tu_b.at[slot], sem_u),
                    pltpu.async_copy(ti_hbm.at[:, src], ti_b.at[slot], sem_i))

        pend = [None, None]
        pend[0] = start(0, 0)
        for ch in range(nch):
            slot = ch % 2
            if ch + 1 < nch:
                pend[(ch + 1) % 2] = start(ch + 1, (ch + 1) % 2)
            du, di = pend[slot]
            du.wait()
            di.wait()

            def g_body(g, carry, slot=slot, ch=ch):
                s0 = pl.multiple_of(g * _L, _L)
                col = pl.ds(s0, _L)
                accu = tu_b[slot, 0, col] * wh[0][0]
                acci = ti_b[slot, 0, col] * wh[2][0]
                for d in range(1, D):
                    accu += tu_b[slot, d, col] * wh[d // _L][d % _L]
                    acci += ti_b[slot, d, col] * wh[2 + d // _L][d % _L]
                su_v[pl.ds(ch * _CL + s0, _L)] = accu
                si_v[pl.ds(ch * _CL + s0, _L)] = acci
                return carry

            lax.fori_loop(0, _CL // _L, g_body, 0)

        pltpu.sync_copy(su_v, su_hbm.at[pl.ds(base, vpw)])
        pltpu.sync_copy(si_v, si_hbm.at[pl.ds(base, vpw)])

    return sc_scores


@functools.lru_cache(maxsize=None)
def _build_sc_gather(B, V, split):
    nw = _NC * _NS
    bpw = B // nw
    n_chunk = bpw // _IDX_CHUNK
    n_grp = bpw // _L

    mesh = plsc.VectorSubcoreMesh(core_axis_name="c", subcore_axis_name="s")

    @functools.partial(
        pl.kernel,
        mesh=mesh,
        compiler_params=pltpu.CompilerParams(
            needs_layout_passes=False, use_tc_tiling_on_sc=False),
        out_type=jax.ShapeDtypeStruct((B,), jnp.float32),
        scratch_types=[
            pltpu.VMEM((n_chunk, _IDX_CHUNK), jnp.int32),
            pltpu.VMEM((n_chunk, _IDX_CHUNK), jnp.int32),
            pltpu.VMEM((n_chunk, _IDX_CHUNK), jnp.int32),
            pltpu.VMEM((n_chunk, _IDX_CHUNK), jnp.int32),
            pltpu.VMEM((n_chunk, _IDX_CHUNK), jnp.int32),
            pltpu.VMEM((n_chunk, _IDX_CHUNK), jnp.int32),
            pltpu.VMEM((bpw,), jnp.float32),
            pltpu.VMEM((bpw,), jnp.float32),
            pltpu.VMEM((bpw,), jnp.float32),
            pltpu.VMEM((bpw,), jnp.float32),
            pltpu.VMEM((_L,), jnp.float32),
            pltpu.VMEM((bpw,), jnp.float32),
            pltpu.SemaphoreType.DMA,
            pltpu.SemaphoreType.DMA,
        ],
    )
    def sc_gather(uidx_hbm, iidx_hbm, su_sc_hbm, si_sc_hbm,
                  su_tc_hbm, si_tc_hbm, b_hbm, out_hbm,
                  uidx_v, iidx_v, usc_v, utc_v, isc_v, itc_v,
                  su_sc_v, su_tc_v, si_sc_v, si_tc_v, b_v, out_v,
                  sem_u, sem_i):
        wid = lax.axis_index("s") * _NC + lax.axis_index("c")
        crow = wid * n_chunk

        pltpu.sync_copy(uidx_hbm.at[pl.ds(crow, n_chunk), :], uidx_v)
        pltpu.sync_copy(iidx_hbm.at[pl.ds(crow, n_chunk), :], iidx_v)
        pltpu.sync_copy(b_hbm, b_v)

        def split_body(q, carry):
            j = q // (_IDX_CHUNK // _L)
            s0 = pl.multiple_of((q % (_IDX_CHUNK // _L)) * _L, _L)
            col = pl.ds(s0, _L)
            for src_v, lo_v, hi_v in ((uidx_v, usc_v, utc_v),
                                      (iidx_v, isc_v, itc_v)):
                iv = src_v[j, col]
                lo_v[j, col] = jnp.minimum(iv, split - 1)
                hi_v[j, col] = jnp.clip(iv - split, 0, V - split - 1)
            return carry

        lax.fori_loop(0, n_chunk * (_IDX_CHUNK // _L), split_body, 0)

        copies = []
        for j in range(n_chunk):
            dst = pl.ds(j * _IDX_CHUNK, _IDX_CHUNK)
            copies.append(pltpu.async_copy(
                su_sc_hbm.at[usc_v.at[j]], su_sc_v.at[dst], sem_u))
            copies.append(pltpu.async_copy(
                su_tc_hbm.at[utc_v.at[j]], su_tc_v.at[dst], sem_u))
            copies.append(pltpu.async_copy(
                si_sc_hbm.at[isc_v.at[j]], si_sc_v.at[dst], sem_i))
            copies.append(pltpu.async_copy(
                si_tc_hbm.at[itc_v.at[j]], si_tc_v.at[dst], sem_i))
        for cp in copies:
            cp.wait()

        bv = b_v[...]

        def grp_body(g, carry):
            s0 = pl.multiple_of(g * _L, _L)
            col = pl.ds(s0, _L)
            j = g // (_IDX_CHUNK // _L)
            jcol = pl.ds((g % (_IDX_CHUNK // _L)) * _L, _L)
            su = jnp.where(uidx_v[j, jcol] < split, su_sc_v[col], su_tc_v[col])
            si = jnp.where(iidx_v[j, jcol] < split, si_sc_v[col], si_tc_v[col])
            x = su + si + bv
            out_v[col] = 1.0 / (1.0 + jnp.exp(-x))
            return carry

        lax.fori_loop(0, n_grp, grp_body, 0)

        pltpu.sync_copy(out_v, out_hbm.at[pl.ds(wid * bpw, bpw)])

    return sc_gather


def kernel(user_input, pos_item_input, user_table, item_table, W, b):
    B = user_input.shape[0]
    V_u, D = user_table.shape
    uidx = user_input.reshape(B // _IDX_CHUNK, _IDX_CHUNK).astype(jnp.int32)
    iidx = pos_item_input.reshape(B // _IDX_CHUNK, _IDX_CHUNK).astype(jnp.int32)
    b16 = jnp.broadcast_to(b.reshape(()), (_L,)).astype(jnp.float32)
    Wf = W.astype(jnp.float32)
    su_sc, si_sc = _build_sc_scores(V_u, D, _SPLIT)(
        user_table.T, item_table.T, Wf.reshape(-1))
    su_tc, si_tc = _build_tc_scores(V_u, D, _BL, _SPLIT)(
        user_table.T, item_table.T, Wf)
    out = _build_sc_gather(B, V_u, _SPLIT)(
        uidx, iidx, su_sc, si_sc, su_tc, si_tc, b16)
    return out.reshape(B, 1)


# speedup vs baseline: 1.6393x; 1.6393x over previous
import functools

import jax
import jax.numpy as jnp
from jax import lax
from jax.experimental import pallas as pl
from jax.experimental.pallas import tpu as pltpu
from jax.experimental.pallas import tpu_sc as plsc

_L = 16
_NC = 2
_NS = 16
_IDX_CHUNK = 128
_BL = 32768


@functools.lru_cache(maxsize=None)
def _build_tc_scores(V, D, bl):
    nb = (V + bl - 1) // bl

    def body(tu_ref, ti_ref, w_ref, ou_ref, oi_ref):
        w = w_ref[...]
        pu = jax.lax.dot_general(
            w[:D], tu_ref[...], (((0,), (0,)), ((), ())),
            preferred_element_type=jnp.float32)
        ou_ref[...] = pu.reshape(ou_ref.shape)
        pi = jax.lax.dot_general(
            w[D:], ti_ref[...], (((0,), (0,)), ((), ())),
            preferred_element_type=jnp.float32)
        oi_ref[...] = pi.reshape(oi_ref.shape)

    return pl.pallas_call(
        body,
        grid=(nb,),
        in_specs=[
            pl.BlockSpec((D, bl), lambda i: (0, i)),
            pl.BlockSpec((D, bl), lambda i: (0, i)),
            pl.BlockSpec((2 * D, 1), lambda i: (0, 0)),
        ],
        out_specs=[
            pl.BlockSpec((bl,), lambda i: (i,)),
            pl.BlockSpec((bl,), lambda i: (i,)),
        ],
        out_shape=[
            jax.ShapeDtypeStruct((V,), jnp.float32),
            jax.ShapeDtypeStruct((V,), jnp.float32),
        ],
        compiler_params=pltpu.CompilerParams(
            dimension_semantics=("parallel",)),
    )


@functools.lru_cache(maxsize=None)
def _build_sc_gather(B):
    nw = _NC * _NS
    bpw = B // nw
    n_chunk = bpw // _IDX_CHUNK
    n_grp = bpw // _L

    mesh = plsc.VectorSubcoreMesh(core_axis_name="c", subcore_axis_name="s")

    @functools.partial(
        pl.kernel,
        mesh=mesh,
        compiler_params=pltpu.CompilerParams(
            needs_layout_passes=False, use_tc_tiling_on_sc=False),
        out_type=jax.ShapeDtypeStruct((B,), jnp.float32),
        scratch_types=[
            pltpu.VMEM((n_chunk, _IDX_CHUNK), jnp.int32),
            pltpu.VMEM((n_chunk, _IDX_CHUNK), jnp.int32),
            pltpu.VMEM((bpw,), jnp.float32),
            pltpu.VMEM((bpw,), jnp.float32),
            pltpu.VMEM((_L,), jnp.float32),
            pltpu.VMEM((bpw,), jnp.float32),
            pltpu.SemaphoreType.DMA,
            pltpu.SemaphoreType.DMA,
        ],
    )
    def sc_kernel(uidx_hbm, iidx_hbm, su_hbm, si_hbm, b_hbm, out_hbm,
                  uidx_v, iidx_v, su_v, si_v, b_v, out_v, sem_u, sem_i):
        wid = lax.axis_index("s") * _NC + lax.axis_index("c")
        crow = wid * n_chunk

        pltpu.sync_copy(uidx_hbm.at[pl.ds(crow, n_chunk), :], uidx_v)
        pltpu.sync_copy(iidx_hbm.at[pl.ds(crow, n_chunk), :], iidx_v)
        pltpu.sync_copy(b_hbm, b_v)

        copies = []
        for j in range(n_chunk):
            dst = pl.ds(j * _IDX_CHUNK, _IDX_CHUNK)
            copies.append(pltpu.async_copy(
                su_hbm.at[uidx_v.at[j]], su_v.at[dst], sem_u))
            copies.append(pltpu.async_copy(
                si_hbm.at[iidx_v.at[j]], si_v.at[dst], sem_i))
        for cp in copies:
            cp.wait()

        bv = b_v[...]

        def grp_body(g, carry):
            s = pl.multiple_of(g * _L, _L)
            x = su_v[pl.ds(s, _L)] + si_v[pl.ds(s, _L)] + bv
            out_v[pl.ds(s, _L)] = 1.0 / (1.0 + jnp.exp(-x))
            return carry

        lax.fori_loop(0, n_grp, grp_body, 0)

        pltpu.sync_copy(out_v, out_hbm.at[pl.ds(wid * bpw, bpw)])

    return sc_kernel


def kernel(user_input, pos_item_input, user_table, item_table, W, b):
    B = user_input.shape[0]
    V_u, D = user_table.shape
    V_i = item_table.shape[0]
    uidx = user_input.reshape(B // _IDX_CHUNK, _IDX_CHUNK).astype(jnp.int32)
    iidx = pos_item_input.reshape(B // _IDX_CHUNK, _IDX_CHUNK).astype(jnp.int32)
    b16 = jnp.broadcast_to(b.reshape(()), (_L,)).astype(jnp.float32)
    scores_u, scores_i = _build_tc_scores(V_u, D, _BL)(
        user_table.T, item_table.T, W.astype(jnp.float32))
    out = _build_sc_gather(B)(uidx, iidx, scores_u, scores_i, b16)
    return out.reshape(B, 1)
